# Initial kernel scaffold; baseline (speedup 1.0000x reference)
import jax
import jax.numpy as jnp
from jax.experimental import pallas as pl


def kernel(candidate_edges, loglog_u, sampled_edges, edges_logits):
    raise NotImplementedError("write your pallas kernel here")


# speedup vs baseline: 69.3995x; 69.3995x over previous
import functools

import jax
import jax.numpy as jnp
from jax import lax
from jax.experimental import pallas as pl
from jax.experimental.pallas import tpu as pltpu
from jax.experimental.pallas import tpu_sc as plsc

N_CAND = 1000000
N_SAMP = 200000
NUM_SEG = 1024

NC, NS = 2, 16
NW = NC * NS
C = 31360
NP = NW * C
S = 6272
NSP = NW * S
NBINS = 1040

_MESH = plsc.VectorSubcoreMesh(core_axis_name="c", subcore_axis_name="s")
_PARAMS = pltpu.CompilerParams(needs_layout_passes=False)


def _wid():
    return lax.axis_index("s") * NC + lax.axis_index("c")


def _pass1_body(eid_hbm, u_hbm, eg_hbm, tab_hbm, e_hbm, pbins_hbm,
                eid_v, u_v, eg_v, e_v, bins_v, sem):
    wid = _wid()
    base = wid * C

    pltpu.sync_copy(eid_hbm.at[pl.ds(base, C)], eid_v)
    pltpu.sync_copy(u_hbm.at[pl.ds(base, C)], u_v)
    pltpu.sync_copy(eg_hbm.at[pl.ds(base, C)], eg_v)

    def zero_bins(i, _):
        bins_v[pl.ds(i * 16, 16)] = jnp.zeros((16,), jnp.float32)
        return _
    lax.fori_loop(0, NBINS // 16, zero_bins, None)

    pltpu.async_copy(tab_hbm.at[eid_v], e_v, sem).wait()

    def step(j, _):
        sl = pl.ds(j * 16, 16)
        e16 = jnp.exp(e_v[sl] + u_v[sl])
        e_v[sl] = e16
        plsc.addupdate_scatter(bins_v, [eg_v[sl]], e16)
        return _
    lax.fori_loop(0, C // 16, step, None)

    pltpu.sync_copy(e_v, e_hbm.at[pl.ds(base, C)])
    pltpu.sync_copy(bins_v, pbins_hbm.at[wid])


def _pass2_body(e_hbm, eg_hbm, pbins_hbm, ca_hbm, y_hbm,
                pb_v, bins_v, ca_v, e_v, seg_v, y_v, sem):
    wid = _wid()
    base = wid * S

    pltpu.sync_copy(ca_hbm.at[pl.ds(base, S)], ca_v)
    pltpu.sync_copy(pbins_hbm, pb_v)

    ge = pltpu.async_copy(e_hbm.at[ca_v], e_v, sem)
    gs = pltpu.async_copy(eg_hbm.at[ca_v], seg_v, sem)

    def red(i, _):
        sl = pl.ds(i * 16, 16)
        acc = pb_v[0, sl]

        def add_row(t, a):
            return a + pb_v[t, sl]
        bins_v[sl] = lax.fori_loop(1, NW, add_row, acc)
        return _
    lax.fori_loop(0, NBINS // 16, red, None)

    ge.wait()
    gs.wait()

    def step(j, _):
        b = j * 16
        denom = plsc.load_gather(bins_v, [seg_v[pl.ds(b, 16)]])
        y = e_v[pl.ds(b, 16)] / denom
        y_v[pl.ds(b, 16)] = (1.0 - y) + y
        return _
    lax.fori_loop(0, S // 16, step, None)

    pltpu.sync_copy(y_v, y_hbm.at[pl.ds(base, S)])


_pass1 = functools.partial(
    pl.kernel,
    out_type=(
        jax.ShapeDtypeStruct((NP,), jnp.float32),
        jax.ShapeDtypeStruct((NW, NBINS), jnp.float32),
    ),
    mesh=_MESH,
    scratch_types=[
        pltpu.VMEM((C,), jnp.int32),
        pltpu.VMEM((C,), jnp.float32),
        pltpu.VMEM((C,), jnp.int32),
        pltpu.VMEM((C,), jnp.float32),
        pltpu.VMEM((NBINS,), jnp.float32),
        pltpu.SemaphoreType.DMA,
    ],
    compiler_params=_PARAMS,
)(_pass1_body)

_pass2 = functools.partial(
    pl.kernel,
    out_type=jax.ShapeDtypeStruct((NSP,), jnp.float32),
    mesh=_MESH,
    scratch_types=[
        pltpu.VMEM((NW, NBINS), jnp.float32),
        pltpu.VMEM((NBINS,), jnp.float32),
        pltpu.VMEM((S,), jnp.int32),
        pltpu.VMEM((S,), jnp.float32),
        pltpu.VMEM((S,), jnp.int32),
        pltpu.VMEM((S,), jnp.float32),
        pltpu.SemaphoreType.DMA,
    ],
    compiler_params=_PARAMS,
)(_pass2_body)


def kernel(candidate_edges, loglog_u, sampled_edges, edges_logits):
    eg = candidate_edges[:, 0]
    eid = candidate_edges[:, 1]
    ca = sampled_edges[:, 5]

    pad = NP - N_CAND
    egp = jnp.concatenate([eg, jnp.full((pad,), NUM_SEG, jnp.int32)])
    eidp = jnp.concatenate([eid, jnp.zeros((pad,), jnp.int32)])
    up = jnp.concatenate([loglog_u, jnp.zeros((pad,), jnp.float32)])
    cap = jnp.concatenate([ca, jnp.zeros((NSP - N_SAMP,), jnp.int32)])

    e, pbins = _pass1(eidp, up, egp, edges_logits)
    ypad = _pass2(e, egp, pbins, cap)
    return ypad[:N_SAMP]


# speedup vs baseline: 91.3854x; 1.3168x over previous
import functools

import jax
import jax.numpy as jnp
from jax import lax
from jax.experimental import pallas as pl
from jax.experimental.pallas import tpu as pltpu
from jax.experimental.pallas import tpu_sc as plsc

N_CAND = 1000000
N_SAMP = 200000
NUM_SEG = 1024

NC, NS = 2, 16
NW = NC * NS
C = 31360
NP = NW * C
NCH = 8
CH = C // NCH
DEPTH = 2
S = 6272
NSP = NW * S
NBINS = 1040

_MESH = plsc.VectorSubcoreMesh(core_axis_name="c", subcore_axis_name="s")
_PARAMS = pltpu.CompilerParams(needs_layout_passes=False)


def _wid():
    return lax.axis_index("s") * NC + lax.axis_index("c")


def _pass1_body(eid_hbm, u_hbm, eg_hbm, tab_hbm, e_hbm, pbins_hbm,
                eid_v, u_v, eg_v, e_v, bins_v,
                sem_a, sem_b, sem_c, gs0, gs1, wsem):
    wid = _wid()
    base = wid * C

    cp_eid = pltpu.async_copy(eid_hbm.at[pl.ds(base, C)], eid_v, sem_a)
    cp_u = pltpu.async_copy(u_hbm.at[pl.ds(base, C)], u_v, sem_b)
    cp_eg = pltpu.async_copy(eg_hbm.at[pl.ds(base, C)], eg_v, sem_c)

    def zero_bins(i, _):
        bins_v[pl.ds(i * 16, 16)] = jnp.zeros((16,), jnp.float32)
        return _
    lax.fori_loop(0, NBINS // 16, zero_bins, None)

    gsems = [gs0, gs1]
    cp_eid.wait()
    g = {}
    for j in range(DEPTH):
        sl = pl.ds(j * CH, CH)
        g[j] = pltpu.async_copy(tab_hbm.at[eid_v.at[sl]], e_v.at[sl],
                                gsems[j % DEPTH])
    cp_u.wait()
    cp_eg.wait()

    lane0 = lax.iota(jnp.int32, 16) == 0
    acc = jnp.zeros((16,), jnp.float32)
    prev = eg_v[pl.ds(0, 16)][0]
    wbs = []
    for c in range(NCH):
        g[c].wait()
        if c + DEPTH < NCH:
            sl = pl.ds((c + DEPTH) * CH, CH)
            g[c + DEPTH] = pltpu.async_copy(tab_hbm.at[eid_v.at[sl]],
                                            e_v.at[sl],
                                            gsems[(c + DEPTH) % DEPTH])

        def step(j, carry, c=c):
            acc, prev = carry
            sl = pl.ds(c * CH + j * 16, 16)
            e16 = jnp.exp(e_v[sl] + u_v[sl])
            e_v[sl] = e16
            seg16 = eg_v[sl]
            s0 = seg16[0]
            s15 = seg16[15]
            uniform = jnp.logical_and(s0 == s15, s0 == prev)
            boundary = s0 != s15

            @pl.when(jnp.logical_not(uniform))
            def _flush():
                w = bins_v[pl.ds(prev, 16)]
                bins_v[pl.ds(prev, 16)] = w + jnp.where(lane0, jnp.sum(acc), 0.0)

            @pl.when(boundary)
            def _scatter():
                plsc.addupdate_scatter(bins_v, [seg16], e16)

            acc_n = jnp.where(uniform, acc + e16,
                              jnp.where(boundary, jnp.zeros_like(e16), e16))
            prev_n = jnp.where(uniform, prev, s15)
            return (acc_n, prev_n)

        acc, prev = lax.fori_loop(0, CH // 16, step, (acc, prev))
        csl = pl.ds(c * CH, CH)
        wbs.append(pltpu.async_copy(e_v.at[csl],
                                    e_hbm.at[pl.ds(base + c * CH, CH)], wsem))

    w = bins_v[pl.ds(prev, 16)]
    bins_v[pl.ds(prev, 16)] = w + jnp.where(lane0, jnp.sum(acc), 0.0)
    pltpu.sync_copy(bins_v, pbins_hbm.at[wid])
    for h in wbs:
        h.wait()


def _pass2_body(e_hbm, eg_hbm, pbins_hbm, ca_hbm, y_hbm,
                pb_v, bins_v, ca_v, e_v, seg_v, y_v, sem_a, sem_b, sem_c, sem_d):
    wid = _wid()
    base = wid * S

    cp_ca = pltpu.async_copy(ca_hbm.at[pl.ds(base, S)], ca_v, sem_a)
    cp_pb = pltpu.async_copy(pbins_hbm, pb_v, sem_b)
    cp_ca.wait()
    ge = pltpu.async_copy(e_hbm.at[ca_v], e_v, sem_c)
    gs = pltpu.async_copy(eg_hbm.at[ca_v], seg_v, sem_d)
    cp_pb.wait()

    def red(i, _):
        sl = pl.ds(i * 16, 16)
        acc = pb_v[0, sl]

        def add_row(t, a):
            return a + pb_v[t, sl]
        bins_v[sl] = lax.fori_loop(1, NW, add_row, acc)
        return _
    lax.fori_loop(0, NBINS // 16, red, None)

    ge.wait()
    gs.wait()

    def step(j, _):
        b = j * 16
        denom = plsc.load_gather(bins_v, [seg_v[pl.ds(b, 16)]])
        y = e_v[pl.ds(b, 16)] / denom
        y_v[pl.ds(b, 16)] = (1.0 - y) + y
        return _
    lax.fori_loop(0, S // 16, step, None)

    pltpu.sync_copy(y_v, y_hbm.at[pl.ds(base, S)])


_pass1 = functools.partial(
    pl.kernel,
    out_type=(
        jax.ShapeDtypeStruct((NP,), jnp.float32),
        jax.ShapeDtypeStruct((NW, NBINS), jnp.float32),
    ),
    mesh=_MESH,
    scratch_types=[
        pltpu.VMEM((C,), jnp.int32),
        pltpu.VMEM((C,), jnp.float32),
        pltpu.VMEM((C,), jnp.int32),
        pltpu.VMEM((C,), jnp.float32),
        pltpu.VMEM((NBINS,), jnp.float32),
        pltpu.SemaphoreType.DMA,
        pltpu.SemaphoreType.DMA,
        pltpu.SemaphoreType.DMA,
        pltpu.SemaphoreType.DMA,
        pltpu.SemaphoreType.DMA,
        pltpu.SemaphoreType.DMA,
    ],
    compiler_params=_PARAMS,
)(_pass1_body)

_pass2 = functools.partial(
    pl.kernel,
    out_type=jax.ShapeDtypeStruct((NSP,), jnp.float32),
    mesh=_MESH,
    scratch_types=[
        pltpu.VMEM((NW, NBINS), jnp.float32),
        pltpu.VMEM((NBINS,), jnp.float32),
        pltpu.VMEM((S,), jnp.int32),
        pltpu.VMEM((S,), jnp.float32),
        pltpu.VMEM((S,), jnp.int32),
        pltpu.VMEM((S,), jnp.float32),
        pltpu.SemaphoreType.DMA,
        pltpu.SemaphoreType.DMA,
        pltpu.SemaphoreType.DMA,
        pltpu.SemaphoreType.DMA,
    ],
    compiler_params=_PARAMS,
)(_pass2_body)


def kernel(candidate_edges, loglog_u, sampled_edges, edges_logits):
    eg = candidate_edges[:, 0]
    eid = candidate_edges[:, 1]
    ca = sampled_edges[:, 5]

    pad = NP - N_CAND
    egp = jnp.concatenate([eg, jnp.full((pad,), NUM_SEG, jnp.int32)])
    eidp = jnp.concatenate([eid, jnp.zeros((pad,), jnp.int32)])
    up = jnp.concatenate([loglog_u, jnp.zeros((pad,), jnp.float32)])
    cap = jnp.concatenate([ca, jnp.zeros((NSP - N_SAMP,), jnp.int32)])

    e, pbins = _pass1(eidp, up, egp, edges_logits)
    ypad = _pass2(e, egp, pbins, cap)
    return ypad[:N_SAMP]
